# baseline (speedup 1.0000x reference)
import jax
import jax.numpy as jnp
from jax.experimental import pallas as pl

_BLOCK_M = 2048
_W = 336
_PW = 384
_NW = 3
_NPAD = _PW * _NW


def _nearest_center_kernel(x_ref, c_ref, xn_ref, cn_ref, out_ref):
    x = x_ref[...]
    c2 = (c_ref[...] * -2.0).astype(jnp.bfloat16)
    dot2 = jax.lax.dot_general(
        x.astype(jnp.bfloat16), c2,
        (((1,), (1,)), ((), ())),
        preferred_element_type=jnp.float32)
    dist = jnp.abs((xn_ref[...] + dot2) + cn_ref[...])
    bm = dist.shape[0]
    d3 = dist.reshape(bm, _NW, _PW)
    wv = jnp.min(d3, axis=-1)
    wi = jnp.argmin(d3, axis=-1).astype(jnp.int32)

    m0, m1, m2 = wv[:, 0], wv[:, 1], wv[:, 2]
    b0 = m0.astype(jnp.bfloat16).astype(jnp.float32)
    win1 = m1 < b0
    acc = jnp.where(win1, m1.astype(jnp.bfloat16).astype(jnp.float32), b0)
    win2 = m2 < acc
    wsel = jnp.where(win2, 2, jnp.where(win1, 1, 0)).astype(jnp.int32)
    local = jnp.where(win2, wi[:, 2], jnp.where(win1, wi[:, 1], wi[:, 0]))
    out_ref[...] = (local + _W * wsel)[:, None]


def kernel(x, centers):
    m, k = x.shape
    n = centers.shape[0]
    x_norm = (x ** 2).sum(-1)[:, None]
    centers_norm = (centers ** 2).sum(-1)

    zpad = jnp.zeros((_PW - _W, k), centers.dtype)
    c_pad = jnp.concatenate([
        centers[0:_W], zpad,
        centers[_W:2 * _W], zpad,
        centers[2 * _W:n], jnp.zeros((_NPAD - 2 * _PW - (n - 2 * _W), k),
                                     centers.dtype)], axis=0)
    inf = jnp.full((_PW - _W,), jnp.inf, jnp.float32)
    cn_pad = jnp.concatenate([
        centers_norm[0:_W], inf,
        centers_norm[_W:2 * _W], inf,
        centers_norm[2 * _W:n],
        jnp.full((_NPAD - 2 * _PW - (n - 2 * _W),), jnp.inf, jnp.float32)],
        axis=0)[None, :]

    grid = m // _BLOCK_M
    out = pl.pallas_call(
        _nearest_center_kernel,
        grid=(grid,),
        in_specs=[
            pl.BlockSpec((_BLOCK_M, k), lambda i: (i, 0)),
            pl.BlockSpec((_NPAD, k), lambda i: (0, 0)),
            pl.BlockSpec((_BLOCK_M, 1), lambda i: (i, 0)),
            pl.BlockSpec((1, _NPAD), lambda i: (0, 0)),
        ],
        out_specs=pl.BlockSpec((_BLOCK_M, 1), lambda i: (i, 0)),
        out_shape=jax.ShapeDtypeStruct((m, 1), jnp.int32),
    )(x, c_pad, x_norm, cn_pad)
    return out.reshape(m)


# speedup vs baseline: 2.5949x; 2.5949x over previous
import jax
import jax.numpy as jnp
from jax.experimental import pallas as pl

_BLOCK_M = 2048
_W = 336
_PW = 384
_NW = 3
_NPAD = _PW * _NW


def _bf(v):
    return v.astype(jnp.bfloat16).astype(jnp.float32)


def _nearest_center_kernel(x_ref, c_ref, xn_ref, cn_ref, out_ref):
    x = x_ref[...]
    c2 = (c_ref[...] * -2.0).astype(jnp.bfloat16)
    dot2 = jax.lax.dot_general(
        x.astype(jnp.bfloat16), c2,
        (((1,), (1,)), ((), ())),
        preferred_element_type=jnp.float32)
    dist = jnp.abs((xn_ref[...] + dot2) + cn_ref[...])
    bm = dist.shape[0]
    lane = jax.lax.broadcasted_iota(jnp.int32, (bm, _NPAD), 1)

    vs = []
    iw = []
    for w in range(_NW):
        dw = dist[:, w * _PW:(w + 1) * _PW]
        vw = jnp.min(dw, axis=1)
        vs.append(vw)
        lw = lane[:, w * _PW:(w + 1) * _PW]
        iw.append(jnp.min(jnp.where(dw == vw[:, None], lw, _NPAD), axis=1))

    b0 = _bf(vs[0])
    win1 = vs[1] < b0
    acc = jnp.where(win1, _bf(vs[1]), b0)
    win2 = vs[2] < acc
    wsel = jnp.where(win2, 2, jnp.where(win1, 1, 0)).astype(jnp.int32)
    pidx = jnp.where(win2, iw[2], jnp.where(win1, iw[1], iw[0]))
    out_ref[...] = (pidx - (_PW - _W) * wsel)[:, None]


def kernel(x, centers):
    m, k = x.shape
    n = centers.shape[0]
    x_norm = (x ** 2).sum(-1)[:, None]
    centers_norm = (centers ** 2).sum(-1)

    zpad = jnp.zeros((_PW - _W, k), centers.dtype)
    c_pad = jnp.concatenate([
        centers[0:_W], zpad,
        centers[_W:2 * _W], zpad,
        centers[2 * _W:n], jnp.zeros((_NPAD - 2 * _PW - (n - 2 * _W), k),
                                     centers.dtype)], axis=0)
    inf = jnp.full((_PW - _W,), jnp.inf, jnp.float32)
    cn_pad = jnp.concatenate([
        centers_norm[0:_W], inf,
        centers_norm[_W:2 * _W], inf,
        centers_norm[2 * _W:n],
        jnp.full((_NPAD - 2 * _PW - (n - 2 * _W),), jnp.inf, jnp.float32)],
        axis=0)[None, :]

    grid = m // _BLOCK_M
    out = pl.pallas_call(
        _nearest_center_kernel,
        grid=(grid,),
        in_specs=[
            pl.BlockSpec((_BLOCK_M, k), lambda i: (i, 0)),
            pl.BlockSpec((_NPAD, k), lambda i: (0, 0)),
            pl.BlockSpec((_BLOCK_M, 1), lambda i: (i, 0)),
            pl.BlockSpec((1, _NPAD), lambda i: (0, 0)),
        ],
        out_specs=pl.BlockSpec((_BLOCK_M, 1), lambda i: (i, 0)),
        out_shape=jax.ShapeDtypeStruct((m, 1), jnp.int32),
    )(x, c_pad, x_norm, cn_pad)
    return out.reshape(m)


# speedup vs baseline: 3.0564x; 1.1778x over previous
import jax
import jax.numpy as jnp
from jax.experimental import pallas as pl

_BLOCK_M = 2048
_W = 336
_PW = 384
_NW = 3
_NPAD = _PW * _NW


def _bf(v):
    return v.astype(jnp.bfloat16).astype(jnp.float32)


def _nearest_center_kernel(x_ref, c_ref, xn_ref, cn_ref, lane_ref, out_ref):
    x = x_ref[...]
    c2 = (c_ref[...] * -2.0).astype(jnp.bfloat16)
    dot2 = jax.lax.dot_general(
        x.astype(jnp.bfloat16), c2,
        (((1,), (1,)), ((), ())),
        preferred_element_type=jnp.float32)
    dist = jnp.abs((xn_ref[...] + dot2) + cn_ref[...])
    lane = lane_ref[...]

    vs = []
    iw = []
    for w in range(_NW):
        dw = dist[:, w * _PW:(w + 1) * _PW]
        vw = jnp.min(dw, axis=1, keepdims=True)
        vs.append(vw[:, 0])
        lw = lane[:, w * _PW:(w + 1) * _PW]
        key = jnp.where(dw == vw, lw, float(_NPAD))
        iw.append(jnp.min(key, axis=1))

    b0 = _bf(vs[0])
    win1 = vs[1] < b0
    acc = jnp.where(win1, _bf(vs[1]), b0)
    win2 = vs[2] < acc
    wsel = jnp.where(win2, 2.0, jnp.where(win1, 1.0, 0.0))
    pidx = jnp.where(win2, iw[2], jnp.where(win1, iw[1], iw[0]))
    out = pidx - (_PW - _W) * wsel
    out_ref[...] = out.astype(jnp.int32)[:, None]


def kernel(x, centers):
    m, k = x.shape
    n = centers.shape[0]
    x_norm = (x ** 2).sum(-1)[:, None]
    centers_norm = (centers ** 2).sum(-1)

    zpad = jnp.zeros((_PW - _W, k), centers.dtype)
    c_pad = jnp.concatenate([
        centers[0:_W], zpad,
        centers[_W:2 * _W], zpad,
        centers[2 * _W:n], jnp.zeros((_NPAD - 2 * _PW - (n - 2 * _W), k),
                                     centers.dtype)], axis=0)
    inf = jnp.full((_PW - _W,), jnp.inf, jnp.float32)
    cn_pad = jnp.concatenate([
        centers_norm[0:_W], inf,
        centers_norm[_W:2 * _W], inf,
        centers_norm[2 * _W:n],
        jnp.full((_NPAD - 2 * _PW - (n - 2 * _W),), jnp.inf, jnp.float32)],
        axis=0)[None, :]
    lane = jnp.arange(_NPAD, dtype=jnp.float32)[None, :]

    grid = m // _BLOCK_M
    out = pl.pallas_call(
        _nearest_center_kernel,
        grid=(grid,),
        in_specs=[
            pl.BlockSpec((_BLOCK_M, k), lambda i: (i, 0)),
            pl.BlockSpec((_NPAD, k), lambda i: (0, 0)),
            pl.BlockSpec((_BLOCK_M, 1), lambda i: (i, 0)),
            pl.BlockSpec((1, _NPAD), lambda i: (0, 0)),
            pl.BlockSpec((1, _NPAD), lambda i: (0, 0)),
        ],
        out_specs=pl.BlockSpec((_BLOCK_M, 1), lambda i: (i, 0)),
        out_shape=jax.ShapeDtypeStruct((m, 1), jnp.int32),
    )(x, c_pad, x_norm, cn_pad, lane)
    return out.reshape(m)


# speedup vs baseline: 3.1182x; 1.0202x over previous
import jax
import jax.numpy as jnp
from jax.experimental import pallas as pl

_BLOCK_M = 2048
_W = 336
_PW = 384
_NW = 3
_NPAD = _PW * _NW


def _bf(v):
    return v.astype(jnp.bfloat16).astype(jnp.float32)


def _nearest_center_kernel(x_ref, c_ref, xn_ref, cn_ref, lane_ref, out_ref):
    x = x_ref[...].astype(jnp.bfloat16)
    c2 = (c_ref[...] * -2.0).astype(jnp.bfloat16)
    xn = xn_ref[...]
    lane = lane_ref[...]

    vs = []
    iw = []
    for w in range(_NW):
        dot2 = jax.lax.dot_general(
            x, c2[w * _PW:(w + 1) * _PW],
            (((1,), (1,)), ((), ())),
            preferred_element_type=jnp.float32)
        dw = jnp.abs((xn + dot2) + cn_ref[:, w * _PW:(w + 1) * _PW])
        vw = jnp.min(dw, axis=1, keepdims=True)
        vs.append(vw[:, 0])
        lw = lane[:, w * _PW:(w + 1) * _PW]
        key = jnp.where(dw == vw, lw, float(_NPAD))
        iw.append(jnp.min(key, axis=1))

    b0 = _bf(vs[0])
    win1 = vs[1] < b0
    acc = jnp.where(win1, _bf(vs[1]), b0)
    win2 = vs[2] < acc
    wsel = jnp.where(win2, 2.0, jnp.where(win1, 1.0, 0.0))
    pidx = jnp.where(win2, iw[2], jnp.where(win1, iw[1], iw[0]))
    out = pidx - (_PW - _W) * wsel
    out_ref[...] = out.astype(jnp.int32)[:, None]


def kernel(x, centers):
    m, k = x.shape
    n = centers.shape[0]
    x_norm = (x ** 2).sum(-1)[:, None]
    centers_norm = (centers ** 2).sum(-1)

    zpad = jnp.zeros((_PW - _W, k), centers.dtype)
    c_pad = jnp.concatenate([
        centers[0:_W], zpad,
        centers[_W:2 * _W], zpad,
        centers[2 * _W:n], jnp.zeros((_NPAD - 2 * _PW - (n - 2 * _W), k),
                                     centers.dtype)], axis=0)
    inf = jnp.full((_PW - _W,), jnp.inf, jnp.float32)
    cn_pad = jnp.concatenate([
        centers_norm[0:_W], inf,
        centers_norm[_W:2 * _W], inf,
        centers_norm[2 * _W:n],
        jnp.full((_NPAD - 2 * _PW - (n - 2 * _W),), jnp.inf, jnp.float32)],
        axis=0)[None, :]
    lane = jnp.arange(_NPAD, dtype=jnp.float32)[None, :]

    grid = m // _BLOCK_M
    out = pl.pallas_call(
        _nearest_center_kernel,
        grid=(grid,),
        in_specs=[
            pl.BlockSpec((_BLOCK_M, k), lambda i: (i, 0)),
            pl.BlockSpec((_NPAD, k), lambda i: (0, 0)),
            pl.BlockSpec((_BLOCK_M, 1), lambda i: (i, 0)),
            pl.BlockSpec((1, _NPAD), lambda i: (0, 0)),
            pl.BlockSpec((1, _NPAD), lambda i: (0, 0)),
        ],
        out_specs=pl.BlockSpec((_BLOCK_M, 1), lambda i: (i, 0)),
        out_shape=jax.ShapeDtypeStruct((m, 1), jnp.int32),
    )(x, c_pad, x_norm, cn_pad, lane)
    return out.reshape(m)
